# baseline (speedup 1.0000x reference)
import functools

import jax
import jax.numpy as jnp
from jax import lax
from jax.experimental import pallas as pl
from jax.experimental.pallas import tpu as pltpu
from jax.experimental.pallas import tpu_sc as plsc

B, S, D = 16, 2048, 256
N = B * S
NC, NS, L = 2, 16, 16
NW = NC * NS
ROWS_PER_W = N // NW
CHUNK = 16
NCHUNK = ROWS_PER_W // CHUNK
VECS = D // L


def _body(recon_hbm, target_hbm, maskf_hbm, out_hbm, rbuf, tbuf, mbuf, obuf):
    wid = lax.axis_index("s") * NC + lax.axis_index("c")
    base = wid * ROWS_PER_W

    def chunk_step(c, carry):
        acc, cnt = carry
        row0 = base + c * CHUNK
        pltpu.sync_copy(recon_hbm.at[pl.ds(row0, CHUNK)], rbuf)
        pltpu.sync_copy(target_hbm.at[pl.ds(row0, CHUNK)], tbuf)
        pltpu.sync_copy(maskf_hbm.at[pl.ds(row0, CHUNK)], mbuf)
        mvec = mbuf[pl.ds(0, L)]
        cnt = cnt + mvec
        for r in range(CHUNK):
            w = mvec[r]
            rowacc = jnp.zeros((L,), jnp.float32)
            for v in range(VECS):
                d = rbuf[r, pl.ds(v * L, L)] - tbuf[r, pl.ds(v * L, L)]
                rowacc = rowacc + d * d
            acc = acc + w * rowacc
        return acc, cnt

    acc, cnt = lax.fori_loop(
        0,
        NCHUNK,
        chunk_step,
        (jnp.zeros((L,), jnp.float32), jnp.zeros((L,), jnp.float32)),
    )
    obuf[pl.ds(0, L)] = acc
    obuf[pl.ds(L, L)] = cnt
    pltpu.sync_copy(obuf, out_hbm.at[wid])


@jax.jit
def kernel(kin_recon, kin_target, mask):
    recon = kin_recon.reshape(N, D)
    target = kin_target.reshape(N, D)
    maskf = mask.reshape(N).astype(jnp.float32)

    mesh = plsc.VectorSubcoreMesh(core_axis_name="c", subcore_axis_name="s")
    out = pl.kernel(
        _body,
        out_type=jax.ShapeDtypeStruct((NW, 2 * L), jnp.float32),
        mesh=mesh,
        scratch_types=[
            pltpu.VMEM((CHUNK, D), jnp.float32),
            pltpu.VMEM((CHUNK, D), jnp.float32),
            pltpu.VMEM((CHUNK,), jnp.float32),
            pltpu.VMEM((2 * L,), jnp.float32),
        ],
    )(recon, target, maskf)

    sums = out[:, :L].sum()
    count = out[:, L:].sum()
    return sums / jnp.maximum(count * D, 1.0)


# speedup vs baseline: 1.3384x; 1.3384x over previous
import jax
import jax.numpy as jnp
from jax import lax
from jax.experimental import pallas as pl
from jax.experimental.pallas import tpu as pltpu
from jax.experimental.pallas import tpu_sc as plsc

B, S, D = 16, 2048, 256
N = B * S
NC, NS, L = 2, 16, 16
NW = NC * NS
ROWS_PER_W = N // NW
CHUNK = 64
NCHUNK = ROWS_PER_W // CHUNK
VECS = D // L
GROUPS = CHUNK // L


def _body(recon_hbm, target_hbm, maskf_hbm, out_hbm,
          rbuf0, rbuf1, tbuf0, tbuf1, mbuf, obuf, sem0, sem1):
    wid = lax.axis_index("s") * NC + lax.axis_index("c")
    base = wid * ROWS_PER_W
    pltpu.sync_copy(maskf_hbm.at[pl.ds(base, ROWS_PER_W)], mbuf)

    rbufs = (rbuf0, rbuf1)
    tbufs = (tbuf0, tbuf1)
    sems = (sem0, sem1)

    def issue(c, b):
        off = (base + c * CHUNK) * D
        pltpu.async_copy(recon_hbm.at[pl.ds(off, CHUNK * D)], rbufs[b], sems[b])
        pltpu.async_copy(target_hbm.at[pl.ds(off, CHUNK * D)], tbufs[b], sems[b])

    def wait(c, b):
        off = (base + c * CHUNK) * D
        pltpu.make_async_copy(
            recon_hbm.at[pl.ds(off, CHUNK * D)], rbufs[b], sems[b]).wait()
        pltpu.make_async_copy(
            target_hbm.at[pl.ds(off, CHUNK * D)], tbufs[b], sems[b]).wait()

    issue(0, 0)
    issue(1, 1)

    def compute_chunk(c, b, acc, cnt):
        rb, tb = rbufs[b], tbufs[b]

        def group(g, carry):
            acc, cnt = carry
            mvec = mbuf[pl.ds(c * CHUNK + g * L, L)]
            cnt = cnt + mvec
            for i in range(L):
                w = mvec[i]
                row0 = (g * L + i) * D
                rowacc = jnp.zeros((L,), jnp.float32)
                for v in range(VECS):
                    d = rb[pl.ds(row0 + v * L, L)] - tb[pl.ds(row0 + v * L, L)]
                    rowacc = rowacc + d * d
                acc = acc + w * rowacc
            return acc, cnt

        return lax.fori_loop(0, GROUPS, group, (acc, cnt))

    def pair_step(p, carry):
        acc, cnt = carry
        for b in range(2):
            c = 2 * p + b
            wait(c, b)

            @pl.when(c < NCHUNK - 2)
            def _():
                issue(c + 2, b)

            acc, cnt = compute_chunk(c, b, acc, cnt)
        return acc, cnt

    acc, cnt = lax.fori_loop(
        0,
        NCHUNK // 2,
        pair_step,
        (jnp.zeros((L,), jnp.float32), jnp.zeros((L,), jnp.float32)),
    )
    obuf[pl.ds(0, L)] = acc
    obuf[pl.ds(L, L)] = cnt
    pltpu.sync_copy(obuf, out_hbm.at[wid])


@jax.jit
def kernel(kin_recon, kin_target, mask):
    recon = kin_recon.reshape(N * D)
    target = kin_target.reshape(N * D)
    maskf = mask.reshape(N).astype(jnp.float32)

    mesh = plsc.VectorSubcoreMesh(core_axis_name="c", subcore_axis_name="s")
    out = pl.kernel(
        _body,
        out_type=jax.ShapeDtypeStruct((NW, 2 * L), jnp.float32),
        mesh=mesh,
        scratch_types=[
            pltpu.VMEM((CHUNK * D,), jnp.float32),
            pltpu.VMEM((CHUNK * D,), jnp.float32),
            pltpu.VMEM((CHUNK * D,), jnp.float32),
            pltpu.VMEM((CHUNK * D,), jnp.float32),
            pltpu.VMEM((ROWS_PER_W,), jnp.float32),
            pltpu.VMEM((2 * L,), jnp.float32),
            pltpu.SemaphoreType.DMA,
            pltpu.SemaphoreType.DMA,
        ],
    )(recon, target, maskf)

    sums = out[:, :L].sum()
    count = out[:, L:].sum()
    return sums / jnp.maximum(count * D, 1.0)


# speedup vs baseline: 1.9577x; 1.4627x over previous
import jax
import jax.numpy as jnp
from jax import lax
from jax.experimental import pallas as pl
from jax.experimental.pallas import tpu as pltpu
from jax.experimental.pallas import tpu_sc as plsc

B, S, D = 16, 2048, 256
N = B * S
NC, NS, L = 2, 16, 16
NW = NC * NS
ROWS_PER_W = N // NW
CHUNK = 64
NCHUNK = ROWS_PER_W // CHUNK
VECS = D // L
GROUPS = CHUNK // L


def _body(recon_hbm, target_hbm, maskf_hbm, out_hbm,
          rbuf0, rbuf1, tbuf0, tbuf1, mbuf, obuf, sem0, sem1):
    wid = lax.axis_index("s") * NC + lax.axis_index("c")
    base = wid * ROWS_PER_W
    pltpu.sync_copy(maskf_hbm.at[pl.ds(base, ROWS_PER_W)],
                    mbuf.at[pl.ds(0, ROWS_PER_W)])

    rbufs = (rbuf0, rbuf1)
    tbufs = (tbuf0, tbuf1)
    sems = (sem0, sem1)

    def issue(c, b):
        off = (base + c * CHUNK) * D
        pltpu.async_copy(recon_hbm.at[pl.ds(off, CHUNK * D)], rbufs[b], sems[b])
        pltpu.async_copy(target_hbm.at[pl.ds(off, CHUNK * D)], tbufs[b], sems[b])

    def wait(c, b):
        off = (base + c * CHUNK) * D
        pltpu.make_async_copy(
            recon_hbm.at[pl.ds(off, CHUNK * D)], rbufs[b], sems[b]).wait()
        pltpu.make_async_copy(
            target_hbm.at[pl.ds(off, CHUNK * D)], tbufs[b], sems[b]).wait()

    issue(0, 0)
    issue(1, 1)

    def compute_chunk(c, b, carry):
        rb, tb = rbufs[b], tbufs[b]

        @plsc.parallel_loop(0, CHUNK, unroll=2, carry=carry)
        def loop(r, carry):
            acc0, acc1, cnt = carry
            w = mbuf[pl.ds(c * CHUNK + r, L)][0]
            cnt = cnt + w
            row0 = r * D
            for v in range(VECS):
                d = rb[pl.ds(row0 + v * L, L)] - tb[pl.ds(row0 + v * L, L)]
                wd = w * d
                if v % 2 == 0:
                    acc0 = acc0 + wd * wd
                else:
                    acc1 = acc1 + wd * wd
            return acc0, acc1, cnt

        return loop

    def pair_step(p, carry):
        for b in range(2):
            c = 2 * p + b
            wait(c, b)

            @pl.when(c < NCHUNK - 2)
            def _():
                issue(c + 2, b)

            carry = compute_chunk(c, b, carry)
        return carry

    acc0, acc1, cnt = lax.fori_loop(
        0,
        NCHUNK // 2,
        pair_step,
        (jnp.zeros((L,), jnp.float32), jnp.zeros((L,), jnp.float32),
         jnp.float32(0.0)),
    )
    obuf[pl.ds(0, L)] = acc0 + acc1
    obuf[pl.ds(L, L)] = jnp.full((L,), cnt, jnp.float32)
    pltpu.sync_copy(obuf, out_hbm.at[wid])


@jax.jit
def kernel(kin_recon, kin_target, mask):
    recon = kin_recon.reshape(N * D)
    target = kin_target.reshape(N * D)
    maskf = mask.reshape(N).astype(jnp.float32)

    mesh = plsc.VectorSubcoreMesh(core_axis_name="c", subcore_axis_name="s")
    out = pl.kernel(
        _body,
        out_type=jax.ShapeDtypeStruct((NW, 2 * L), jnp.float32),
        mesh=mesh,
        scratch_types=[
            pltpu.VMEM((CHUNK * D,), jnp.float32),
            pltpu.VMEM((CHUNK * D,), jnp.float32),
            pltpu.VMEM((CHUNK * D,), jnp.float32),
            pltpu.VMEM((CHUNK * D,), jnp.float32),
            pltpu.VMEM((ROWS_PER_W + L,), jnp.float32),
            pltpu.VMEM((2 * L,), jnp.float32),
            pltpu.SemaphoreType.DMA,
            pltpu.SemaphoreType.DMA,
        ],
    )(recon, target, maskf)

    sums = out[:, :L].sum()
    count = out[:, L].sum()
    return sums / jnp.maximum(count * D, 1.0)


# speedup vs baseline: 4.2283x; 2.1598x over previous
import jax
import jax.numpy as jnp
from jax import lax
from jax.experimental import pallas as pl
from jax.experimental.pallas import tpu as pltpu
from jax.experimental.pallas import tpu_sc as plsc

B, S, D = 16, 2048, 256
N = B * S
NC, NS, L = 2, 16, 16
NW = NC * NS
ROWS_PER_W = N // NW
CHUNK = 64
NCHUNK = ROWS_PER_W // CHUNK
VECS = D // L
GROUPS = CHUNK // L


def _body(recon_hbm, target_hbm, maskf_hbm, out_hbm,
          rbuf0, rbuf1, tbuf0, tbuf1, mbuf, obuf, sem0, sem1):
    wid = lax.axis_index("s") * NC + lax.axis_index("c")
    base = wid * ROWS_PER_W
    pltpu.sync_copy(maskf_hbm.at[pl.ds(base, ROWS_PER_W)],
                    mbuf.at[pl.ds(0, ROWS_PER_W)])

    rbufs = (rbuf0, rbuf1)
    tbufs = (tbuf0, tbuf1)
    sems = (sem0, sem1)

    def issue(c, b):
        row0 = base + c * CHUNK
        pltpu.async_copy(recon_hbm.at[pl.ds(row0, CHUNK)], rbufs[b], sems[b])
        pltpu.async_copy(target_hbm.at[pl.ds(row0, CHUNK)], tbufs[b], sems[b])

    def wait(c, b):
        row0 = base + c * CHUNK
        pltpu.make_async_copy(
            recon_hbm.at[pl.ds(row0, CHUNK)], rbufs[b], sems[b]).wait()
        pltpu.make_async_copy(
            target_hbm.at[pl.ds(row0, CHUNK)], tbufs[b], sems[b]).wait()

    issue(0, 0)
    issue(1, 1)

    def compute_chunk(c, b, carry):
        rb, tb = rbufs[b], tbufs[b]

        @plsc.parallel_loop(0, CHUNK, unroll=2, carry=carry)
        def loop(r, carry):
            acc0, acc1, cnt = carry
            w = mbuf[pl.ds(c * CHUNK + r, L)][0]
            cnt = cnt + w
            for v in range(VECS):
                d = rb[r, pl.ds(v * L, L)] - tb[r, pl.ds(v * L, L)]
                wd = w * d
                if v % 2 == 0:
                    acc0 = acc0 + wd * wd
                else:
                    acc1 = acc1 + wd * wd
            return acc0, acc1, cnt

        return loop

    def pair_step(p, carry):
        for b in range(2):
            c = 2 * p + b
            wait(c, b)

            @pl.when(c < NCHUNK - 2)
            def _():
                issue(c + 2, b)

            carry = compute_chunk(c, b, carry)
        return carry

    acc0, acc1, cnt = lax.fori_loop(
        0,
        NCHUNK // 2,
        pair_step,
        (jnp.zeros((L,), jnp.float32), jnp.zeros((L,), jnp.float32),
         jnp.float32(0.0)),
    )
    obuf[pl.ds(0, L)] = acc0 + acc1
    obuf[pl.ds(L, L)] = jnp.full((L,), cnt, jnp.float32)
    pltpu.sync_copy(obuf, out_hbm.at[wid])


@jax.jit
def kernel(kin_recon, kin_target, mask):
    recon = kin_recon.reshape(N, D)
    target = kin_target.reshape(N, D)
    maskf = mask.reshape(N).astype(jnp.float32)

    mesh = plsc.VectorSubcoreMesh(core_axis_name="c", subcore_axis_name="s")
    out = pl.kernel(
        _body,
        out_type=jax.ShapeDtypeStruct((NW, 2 * L), jnp.float32),
        mesh=mesh,
        compiler_params=pltpu.CompilerParams(use_tc_tiling_on_sc=True),
        scratch_types=[
            pltpu.VMEM((CHUNK, D), jnp.float32),
            pltpu.VMEM((CHUNK, D), jnp.float32),
            pltpu.VMEM((CHUNK, D), jnp.float32),
            pltpu.VMEM((CHUNK, D), jnp.float32),
            pltpu.VMEM((ROWS_PER_W + L,), jnp.float32),
            pltpu.VMEM((2 * L,), jnp.float32),
            pltpu.SemaphoreType.DMA,
            pltpu.SemaphoreType.DMA,
        ],
    )(recon, target, maskf)

    sums = out[:, :L].sum()
    count = out[:, L].sum()
    return sums / jnp.maximum(count * D, 1.0)
